# baseline (speedup 1.0000x reference)
import functools

import jax
import jax.numpy as jnp
from jax import lax
from jax.experimental import pallas as pl
from jax.experimental.pallas import tpu as pltpu
from jax.experimental.pallas import tpu_sc as plsc

E = 3_200_000
D = 16
V_DIR, V_TYPE, V_RING = 12, 27, 7
NV = V_DIR + V_TYPE + V_RING
NT = V_DIR * V_TYPE * V_RING
NC, NS = 2, 16
NW = NC * NS
EPW = E // NW
CHUNK = 160
NCHUNK = EPW // CHUNK
GROUPS = CHUNK // 16
NBUF = 4
LOOKAHEAD = NBUF - 1

_idx_buf = lambda: pltpu.VMEM((CHUNK,), jnp.int32)
_row_buf = lambda: pltpu.VMEM((CHUNK, D), jnp.float32)


@functools.partial(
    pl.kernel,
    mesh=plsc.VectorSubcoreMesh(core_axis_name="c", subcore_axis_name="s"),
    out_type=jax.ShapeDtypeStruct((E, D), jnp.float32),
    scratch_types=(
        [pltpu.VMEM((NV * D,), jnp.float32),
         pltpu.VMEM((NT * D,), jnp.float32)]
        + [_idx_buf() for _ in range(3 * NBUF)]
        + [_row_buf() for _ in range(NBUF)]
        + [pltpu.SemaphoreType.DMA for _ in range(2 * NBUF)]
    ),
)
def _sc_lookup(dir_hbm, type_hbm, ring_hbm, w_hbm, out_hbm, wv, tv, *ring):
    idxb = ring[:3 * NBUF]
    rowb = ring[3 * NBUF:4 * NBUF]
    semin = ring[4 * NBUF:5 * NBUF]
    semout = ring[5 * NBUF:6 * NBUF]
    bufs = tuple(
        (idxb[3 * b], idxb[3 * b + 1], idxb[3 * b + 2],
         rowb[b], semin[b], semout[b])
        for b in range(NBUF)
    )

    wid = lax.axis_index("s") * NC + lax.axis_index("c")
    tbase = wid * EPW

    pltpu.sync_copy(w_hbm, wv)

    def build_body(r, _):
        i = r // (V_TYPE * V_RING)
        rem = r - i * (V_TYPE * V_RING)
        j = rem // V_RING
        k = rem - j * V_RING
        tv[pl.ds(r * D, D)] = ((wv[pl.ds(i * D, D)]
                                + wv[pl.ds((V_DIR + j) * D, D)])
                               + wv[pl.ds((V_DIR + V_TYPE + k) * D, D)])
        return 0

    lax.fori_loop(0, NT, build_body, 0)

    def in_descs(ci, b):
        db, tb, rb, _, s, _ = bufs[b]
        base = pl.multiple_of(tbase + ci * CHUNK, 8)
        return ((dir_hbm.at[pl.ds(base, CHUNK)], db, s),
                (type_hbm.at[pl.ds(base, CHUNK)], tb, s),
                (ring_hbm.at[pl.ds(base, CHUNK)], rb, s))

    def out_desc(ci, b):
        rw, so = bufs[b][3], bufs[b][5]
        base = pl.multiple_of(tbase + ci * CHUNK, 8)
        return (rw, out_hbm.at[pl.ds(base, CHUNK)], so)

    def compute(b):
        db, tb, rb, rw = bufs[b][:4]

        @plsc.parallel_loop(0, GROUPS, unroll=2)
        def group_body(g):
            e0 = g * 16
            cv = (db[pl.ds(e0, 16)] * (V_TYPE * V_RING)
                  + tb[pl.ds(e0, 16)] * V_RING
                  + rb[pl.ds(e0, 16)]) * D
            for u in range(16):
                rw[e0 + u] = tv[pl.ds(cv[u], D)]

    def step(ci, b, guarded):
        nci = ci + LOOKAHEAD

        @pl.when(nci < NCHUNK)
        def _():
            for desc in in_descs(nci, (b + LOOKAHEAD) % NBUF):
                pltpu.async_copy(*desc)

        for desc in in_descs(ci, b):
            pltpu.make_async_copy(*desc).wait()

        if guarded:
            @pl.when(ci >= NBUF)
            def _():
                pltpu.make_async_copy(*out_desc(ci, b)).wait()
        else:
            pltpu.make_async_copy(*out_desc(ci, b)).wait()

        compute(b)
        pltpu.async_copy(*out_desc(ci, b))

    for ci in range(LOOKAHEAD):
        for desc in in_descs(ci, ci % NBUF):
            pltpu.async_copy(*desc)

    def quad_body(p, _):
        for b in range(NBUF):
            step(p * NBUF + b, b, guarded=True)
        return 0

    lax.fori_loop(0, NCHUNK // NBUF, quad_body, 0)

    for t in range(NCHUNK % NBUF):
        step(NCHUNK - NCHUNK % NBUF + t, t, guarded=True)

    for t in range(NBUF):
        ci = NCHUNK - NBUF + t
        pltpu.make_async_copy(*out_desc(ci, ci % NBUF)).wait()


def kernel(bond_dir, bond_type, is_in_ring, W_bond_dir, W_bond_type, W_is_in_ring):
    wflat = jnp.concatenate([W_bond_dir.reshape(-1),
                             W_bond_type.reshape(-1),
                             W_is_in_ring.reshape(-1)])
    return _sc_lookup(bond_dir, bond_type, is_in_ring, wflat)


# speedup vs baseline: 1.0196x; 1.0196x over previous
import functools

import jax
import jax.numpy as jnp
from jax import lax
from jax.experimental import pallas as pl
from jax.experimental.pallas import tpu as pltpu
from jax.experimental.pallas import tpu_sc as plsc

E = 3_200_000
D = 16
V_DIR, V_TYPE, V_RING = 12, 27, 7
NV = V_DIR + V_TYPE + V_RING
NT2 = V_DIR * V_TYPE
NC, NS = 2, 16
NW = NC * NS
EPW = E // NW
CHUNK = 400
NCHUNK = EPW // CHUNK
GROUPS = CHUNK // 16


@functools.partial(
    pl.kernel,
    mesh=plsc.VectorSubcoreMesh(core_axis_name="c", subcore_axis_name="s"),
    out_type=jax.ShapeDtypeStruct((E, D), jnp.float32),
    scratch_types=[
        pltpu.VMEM((NV * D,), jnp.float32),
        pltpu.VMEM((NT2 * D,), jnp.float32),
        pltpu.VMEM((CHUNK,), jnp.int32),
        pltpu.VMEM((CHUNK,), jnp.int32),
        pltpu.VMEM((CHUNK,), jnp.int32),
        pltpu.VMEM((CHUNK,), jnp.int32),
        pltpu.VMEM((CHUNK,), jnp.int32),
        pltpu.VMEM((CHUNK,), jnp.int32),
        pltpu.VMEM((CHUNK, D), jnp.float32),
        pltpu.VMEM((CHUNK, D), jnp.float32),
        pltpu.SemaphoreType.DMA,
        pltpu.SemaphoreType.DMA,
        pltpu.SemaphoreType.DMA,
        pltpu.SemaphoreType.DMA,
    ],
)
def _sc_lookup(dir_hbm, type_hbm, ring_hbm, w_hbm, out_hbm,
               wv, tv,
               dirb0, typeb0, ringb0, dirb1, typeb1, ringb1,
               rows0, rows1, semin0, semin1, semout0, semout1):
    wid = lax.axis_index("s") * NC + lax.axis_index("c")
    tbase = wid * EPW

    pltpu.sync_copy(w_hbm, wv)

    def build_body(r, _):
        i = r // V_TYPE
        j = r - i * V_TYPE
        tv[pl.ds(r * D, D)] = (wv[pl.ds(i * D, D)]
                               + wv[pl.ds((V_DIR + j) * D, D)])
        return 0

    lax.fori_loop(0, NT2, build_body, 0)

    bufs = ((dirb0, typeb0, ringb0, rows0, semin0, semout0),
            (dirb1, typeb1, ringb1, rows1, semin1, semout1))

    def in_descs(ci, db, tb, rb, s):
        base = pl.multiple_of(tbase + ci * CHUNK, 8)
        return ((dir_hbm.at[pl.ds(base, CHUNK)], db, s),
                (type_hbm.at[pl.ds(base, CHUNK)], tb, s),
                (ring_hbm.at[pl.ds(base, CHUNK)], rb, s))

    def out_desc(ci, rw, s):
        base = pl.multiple_of(tbase + ci * CHUNK, 8)
        return (rw, out_hbm.at[pl.ds(base, CHUNK)], s)

    def compute(db, tb, rb, rw):
        @plsc.parallel_loop(0, GROUPS, unroll=2)
        def group_body(g):
            e0 = g * 16
            cv = (db[pl.ds(e0, 16)] * V_TYPE + tb[pl.ds(e0, 16)]) * D
            rv = (rb[pl.ds(e0, 16)] + (V_DIR + V_TYPE)) * D
            for u in range(16):
                rw[e0 + u] = tv[pl.ds(cv[u], D)] + wv[pl.ds(rv[u], D)]

    for desc in in_descs(0, dirb0, typeb0, ringb0, semin0):
        pltpu.async_copy(*desc)

    def pair_body(p, _):
        for b in range(2):
            db, tb, rb, rw, si, so = bufs[b]
            odb, otb, orb, _, osi, _ = bufs[1 - b]
            ci = p * 2 + b
            nci = ci + 1

            @pl.when(nci < NCHUNK)
            def _():
                for desc in in_descs(nci, odb, otb, orb, osi):
                    pltpu.async_copy(*desc)

            for desc in in_descs(ci, db, tb, rb, si):
                pltpu.make_async_copy(*desc).wait()

            @pl.when(ci >= 2)
            def _():
                pltpu.make_async_copy(*out_desc(ci, rw, so)).wait()

            compute(db, tb, rb, rw)
            pltpu.async_copy(*out_desc(ci, rw, so))
        return 0

    lax.fori_loop(0, NCHUNK // 2, pair_body, 0)

    pltpu.make_async_copy(*out_desc(NCHUNK - 2, rows0, semout0)).wait()
    pltpu.make_async_copy(*out_desc(NCHUNK - 1, rows1, semout1)).wait()


def kernel(bond_dir, bond_type, is_in_ring, W_bond_dir, W_bond_type, W_is_in_ring):
    wflat = jnp.concatenate([W_bond_dir.reshape(-1),
                             W_bond_type.reshape(-1),
                             W_is_in_ring.reshape(-1)])
    return _sc_lookup(bond_dir, bond_type, is_in_ring, wflat)
